# baseline (speedup 1.0000x reference)
import functools

import numpy as np
import jax
import jax.numpy as jnp
from jax.experimental import pallas as pl
from jax.experimental.pallas import tpu as pltpu

_BN_EPS = 1e-5


def _conv_image(xs_b, w_ref, K, Ho):
    kin = xs_b.shape[1]
    acc = None
    for kh in range(K):
        a = jax.lax.slice(xs_b, (kh, 0), (kh + Ho, kin))
        m = jnp.dot(a, w_ref[kh], preferred_element_type=jnp.float32)
        acc = m if acc is None else acc + m
    return acc


def _stats_kernel(xb_ref, w_ref, s_ref, *, K, Ho, B):
    for b in range(B):
        acc = _conv_image(xb_ref[b], w_ref, K, Ho)
        s0 = jnp.sum(acc, axis=0, keepdims=True)
        s1 = jnp.sum(acc * acc, axis=0, keepdims=True)
        s_ref[b] = jnp.concatenate([s0, s1], axis=0)


def _affine_kernel(xb_ref, w_ref, sc_ref, sh_ref, o_ref, *, K, Ho, B,
                   Cout, Wo):
    for b in range(B):
        acc = _conv_image(xb_ref[b], w_ref, K, Ho)
        y = jnp.maximum(acc * sc_ref[...] + sh_ref[...], 0.0)
        for co in range(Cout):
            o_ref[b, co] = jax.lax.slice(y, (0, co * Wo), (Ho, (co + 1) * Wo))


def _toeplitz(weight, *, stride, padding, K, Cin, Cout, W, Wo, kin, lane):
    wtr = jnp.transpose(weight, (2, 3, 1, 0))
    kh_i, kw_i, ci_i, wo_i, co_i = np.meshgrid(
        np.arange(K), np.arange(K), np.arange(Cin), np.arange(Wo),
        np.arange(Cout), indexing="ij")
    w_in = wo_i * stride + kw_i - padding
    valid = (w_in >= 0) & (w_in < W)
    w3 = jnp.zeros((K, kin, lane), jnp.float32)
    w3 = w3.at[kh_i[valid], (ci_i * W + w_in)[valid],
               (co_i * Wo + wo_i)[valid]].set(
        wtr[kh_i[valid], kw_i[valid], ci_i[valid], co_i[valid]])
    return w3


def _conv_bn_relu(x, weight, gamma, beta, *, stride, padding):
    N, Cin, H, W = x.shape
    Cout, _, K, _ = weight.shape
    Ho = (H + 2 * padding - K) // stride + 1
    Wo = (W + 2 * padding - K) // stride + 1
    lane = Cout * Wo
    kin = Cin * W
    Hpad = -(-(Ho + 2) // 16) * 16

    xh = jnp.transpose(x, (0, 2, 1, 3))
    xh = jnp.pad(xh, ((0, 0), (padding, Hpad - H - padding), (0, 0), (0, 0)))
    xb = xh.reshape(N, Hpad, kin)

    w3 = _toeplitz(weight, stride=stride, padding=padding, K=K, Cin=Cin,
                   Cout=Cout, W=W, Wo=Wo, kin=kin, lane=lane)

    fl = 2 * N * K * Ho * kin * lane
    B = 8 if N % 8 == 0 else 1

    pass1 = pl.pallas_call(
        functools.partial(_stats_kernel, K=K, Ho=Ho, B=B),
        grid=(N // B,),
        in_specs=[
            pl.BlockSpec((B, Hpad, kin), lambda i: (i, 0, 0)),
            pl.BlockSpec((K, kin, lane), lambda i: (0, 0, 0)),
        ],
        out_specs=pl.BlockSpec((B, 2, lane), lambda i: (i, 0, 0)),
        out_shape=jax.ShapeDtypeStruct((N, 2, lane), jnp.float32),
        compiler_params=pltpu.CompilerParams(
            dimension_semantics=("parallel",)),
        cost_estimate=pl.CostEstimate(
            flops=fl, transcendentals=0,
            bytes_accessed=N * Hpad * kin * 4 + N * 2 * lane * 4),
    )
    part = pass1(xb, w3)

    Mtot = N * Ho * Wo
    st = part.reshape(N, 2, Cout, Wo).sum(axis=(0, 3))
    mean = st[0] / Mtot
    var = st[1] / Mtot - mean * mean
    scale_c = gamma.astype(jnp.float32) * jax.lax.rsqrt(var + _BN_EPS)
    shift_c = beta.astype(jnp.float32) - mean * scale_c
    scale_l = jnp.repeat(scale_c, Wo).reshape(1, lane)
    shift_l = jnp.repeat(shift_c, Wo).reshape(1, lane)

    B2 = 2 if N % 2 == 0 else 1
    pass2 = pl.pallas_call(
        functools.partial(_affine_kernel, K=K, Ho=Ho, B=B2, Cout=Cout, Wo=Wo),
        grid=(N // B2,),
        in_specs=[
            pl.BlockSpec((B2, Hpad, kin), lambda i: (i, 0, 0)),
            pl.BlockSpec((K, kin, lane), lambda i: (0, 0, 0)),
            pl.BlockSpec((1, lane), lambda i: (0, 0)),
            pl.BlockSpec((1, lane), lambda i: (0, 0)),
        ],
        out_specs=pl.BlockSpec((B2, Cout, Ho, Wo), lambda i: (i, 0, 0, 0)),
        out_shape=jax.ShapeDtypeStruct((N, Cout, Ho, Wo), jnp.float32),
        compiler_params=pltpu.CompilerParams(
            dimension_semantics=("parallel",)),
        cost_estimate=pl.CostEstimate(
            flops=fl + 3 * N * Ho * lane, transcendentals=0,
            bytes_accessed=(N * Hpad * kin + N * Cout * Ho * Wo) * 4),
    )
    return pass2(xb, w3, scale_l, shift_l).astype(x.dtype)


def kernel(x, weight, bias, gamma, beta):
    del bias
    return _conv_bn_relu(x, weight, gamma, beta, stride=1, padding=1)


# speedup vs baseline: 2.1766x; 2.1766x over previous
import functools

import numpy as np
import jax
import jax.numpy as jnp
from jax.experimental import pallas as pl
from jax.experimental.pallas import tpu as pltpu

_BN_EPS = 1e-5


def _conv_image(xs_b, w_ref, K, Ho):
    kin = xs_b.shape[1]
    acc = None
    for kh in range(K):
        a = jax.lax.slice(xs_b, (kh, 0), (kh + Ho, kin))
        m = jnp.dot(a, w_ref[kh], preferred_element_type=jnp.float32)
        acc = m if acc is None else acc + m
    return acc


def _stats_kernel(xb_ref, w_ref, s_ref, *, K, Ho, B):
    for b in range(B):
        acc = _conv_image(xb_ref[b], w_ref, K, Ho)
        s0 = jnp.sum(acc, axis=0, keepdims=True)
        s1 = jnp.sum(acc * acc, axis=0, keepdims=True)
        s_ref[b] = jnp.concatenate([s0, s1], axis=0)


def _affine_kernel(xb_ref, w_ref, sc_ref, sh_ref, o_ref, *, K, Ho, B):
    for b in range(B):
        acc = _conv_image(xb_ref[b], w_ref, K, Ho)
        o_ref[b] = jnp.maximum(acc * sc_ref[...] + sh_ref[...], 0.0)


def _toeplitz(weight, *, stride, padding, K, Cin, Cout, W, Wo, kin, lane):
    wtr = jnp.transpose(weight, (2, 3, 1, 0))
    kh_i, kw_i, ci_i, wo_i, co_i = np.meshgrid(
        np.arange(K), np.arange(K), np.arange(Cin), np.arange(Wo),
        np.arange(Cout), indexing="ij")
    w_in = wo_i * stride + kw_i - padding
    valid = (w_in >= 0) & (w_in < W)
    w3 = jnp.zeros((K, kin, lane), jnp.float32)
    w3 = w3.at[kh_i[valid], (ci_i * W + w_in)[valid],
               (co_i * Wo + wo_i)[valid]].set(
        wtr[kh_i[valid], kw_i[valid], ci_i[valid], co_i[valid]])
    return w3


def _conv_bn_relu(x, weight, gamma, beta, *, stride, padding):
    N, Cin, H, W = x.shape
    Cout, _, K, _ = weight.shape
    Ho = (H + 2 * padding - K) // stride + 1
    Wo = (W + 2 * padding - K) // stride + 1
    lane = Cout * Wo
    kin = Cin * W
    Hpad = -(-(Ho + 2) // 16) * 16

    xh = jnp.transpose(x, (0, 2, 1, 3))
    xh = jnp.pad(xh, ((0, 0), (padding, Hpad - H - padding), (0, 0), (0, 0)))
    xb = xh.reshape(N, Hpad, kin)

    w3 = _toeplitz(weight, stride=stride, padding=padding, K=K, Cin=Cin,
                   Cout=Cout, W=W, Wo=Wo, kin=kin, lane=lane)

    fl = 2 * N * K * Ho * kin * lane
    B = 8 if N % 8 == 0 else 1

    pass1 = pl.pallas_call(
        functools.partial(_stats_kernel, K=K, Ho=Ho, B=B),
        grid=(N // B,),
        in_specs=[
            pl.BlockSpec((B, Hpad, kin), lambda i: (i, 0, 0)),
            pl.BlockSpec((K, kin, lane), lambda i: (0, 0, 0)),
        ],
        out_specs=pl.BlockSpec((B, 2, lane), lambda i: (i, 0, 0)),
        out_shape=jax.ShapeDtypeStruct((N, 2, lane), jnp.float32),
        compiler_params=pltpu.CompilerParams(
            dimension_semantics=("parallel",)),
        cost_estimate=pl.CostEstimate(
            flops=fl, transcendentals=0,
            bytes_accessed=N * Hpad * kin * 4 + N * 2 * lane * 4),
    )
    part = pass1(xb, w3)

    Mtot = N * Ho * Wo
    st = part.reshape(N, 2, Cout, Wo).sum(axis=(0, 3))
    mean = st[0] / Mtot
    var = st[1] / Mtot - mean * mean
    scale_c = gamma.astype(jnp.float32) * jax.lax.rsqrt(var + _BN_EPS)
    shift_c = beta.astype(jnp.float32) - mean * scale_c
    scale_l = jnp.repeat(scale_c, Wo).reshape(1, lane)
    shift_l = jnp.repeat(shift_c, Wo).reshape(1, lane)

    pass2 = pl.pallas_call(
        functools.partial(_affine_kernel, K=K, Ho=Ho, B=B),
        grid=(N // B,),
        in_specs=[
            pl.BlockSpec((B, Hpad, kin), lambda i: (i, 0, 0)),
            pl.BlockSpec((K, kin, lane), lambda i: (0, 0, 0)),
            pl.BlockSpec((1, lane), lambda i: (0, 0)),
            pl.BlockSpec((1, lane), lambda i: (0, 0)),
        ],
        out_specs=pl.BlockSpec((B, Ho, lane), lambda i: (i, 0, 0)),
        out_shape=jax.ShapeDtypeStruct((N, Ho, lane), jnp.float32),
        compiler_params=pltpu.CompilerParams(
            dimension_semantics=("parallel",)),
        cost_estimate=pl.CostEstimate(
            flops=fl + 3 * N * Ho * lane, transcendentals=0,
            bytes_accessed=(N * Hpad * kin + N * Ho * lane) * 4),
    )
    y = pass2(xb, w3, scale_l, shift_l)

    out = y.reshape(N, Ho, Cout, Wo)
    return jnp.transpose(out, (0, 2, 1, 3)).astype(x.dtype)


def kernel(x, weight, bias, gamma, beta):
    del bias
    return _conv_bn_relu(x, weight, gamma, beta, stride=1, padding=1)


# speedup vs baseline: 2.8312x; 1.3008x over previous
import functools

import numpy as np
import jax
import jax.numpy as jnp
from jax.experimental import pallas as pl
from jax.experimental.pallas import tpu as pltpu

_BN_EPS = 1e-5


def _conv_image(xb, w_ref, K, Ho):
    kin = xb.shape[1]
    acc = None
    for kh in range(K):
        a = jax.lax.slice(xb, (kh, 0), (kh + Ho, kin))
        m = jnp.dot(a, w_ref[kh], preferred_element_type=jnp.float32)
        acc = m if acc is None else acc + m
    return acc


def _stats_kernel(xb_ref, w_ref, s_ref, *, K, Ho, B):
    for b in range(B):
        acc = _conv_image(xb_ref[b], w_ref, K, Ho)
        s0 = jnp.sum(acc, axis=0, keepdims=True)
        s1 = jnp.sum(acc * acc, axis=0, keepdims=True)
        s_ref[b] = jnp.concatenate([s0, s1], axis=0)


def _affine_kernel(xb_ref, w_ref, sc_ref, sh_ref, o_ref, *, K, Ho, B):
    for b in range(B):
        acc = _conv_image(xb_ref[b], w_ref, K, Ho)
        o_ref[b] = jnp.maximum(acc * sc_ref[...] + sh_ref[...], 0.0)


def _conv_bn_relu(x, weight, gamma, beta, *, stride, padding):
    N, Cin, H, W = x.shape
    Cout, _, K, _ = weight.shape
    Ho = (H + 2 * padding - K) // stride + 1
    Wo = (W + 2 * padding - K) // stride + 1
    Wp = W + 2 * padding
    lane = Cout * Wo
    kin = Wp * Cin

    rows_needed = (Ho - 1) * stride + K
    Hpad = -(-rows_needed // 16) * 16
    xh = jnp.transpose(x, (0, 2, 3, 1))
    xh = jnp.pad(xh, ((0, 0), (padding, Hpad - H - padding),
                      (padding, padding), (0, 0)))
    xb = xh.reshape(N, Hpad, kin).astype(jnp.bfloat16)

    wt = jnp.transpose(weight, (2, 3, 1, 0))
    kw_i, ci_i, wo_i, co_i = np.meshgrid(
        np.arange(K), np.arange(Cin), np.arange(Wo), np.arange(Cout),
        indexing="ij")
    rows = (wo_i * stride + kw_i) * Cin + ci_i
    cols = co_i * Wo + wo_i
    w3 = jnp.zeros((K, kin, lane), jnp.float32)
    w3 = w3.at[:, rows, cols].set(wt[:, kw_i, ci_i, co_i]).astype(jnp.bfloat16)

    fl = 2 * N * K * Ho * kin * lane
    itemsize = 2
    B = 8 if N % 8 == 0 else 1

    pass1 = pl.pallas_call(
        functools.partial(_stats_kernel, K=K, Ho=Ho, B=B),
        grid=(N // B,),
        in_specs=[
            pl.BlockSpec((B, Hpad, kin), lambda i: (i, 0, 0)),
            pl.BlockSpec((K, kin, lane), lambda i: (0, 0, 0)),
        ],
        out_specs=pl.BlockSpec((B, 2, lane), lambda i: (i, 0, 0)),
        out_shape=jax.ShapeDtypeStruct((N, 2, lane), jnp.float32),
        compiler_params=pltpu.CompilerParams(
            dimension_semantics=("parallel",)),
        cost_estimate=pl.CostEstimate(
            flops=fl, transcendentals=0,
            bytes_accessed=N * Hpad * kin * itemsize + N * 2 * lane * 4),
    )
    part = pass1(xb, w3)

    Mtot = N * Ho * Wo
    st = part.reshape(N, 2, Cout, Wo).sum(axis=(0, 3))
    mean = st[0] / Mtot
    var = st[1] / Mtot - mean * mean
    scale_c = gamma.astype(jnp.float32) * jax.lax.rsqrt(var + _BN_EPS)
    shift_c = beta.astype(jnp.float32) - mean * scale_c
    scale_l = jnp.repeat(scale_c, Wo).reshape(1, lane)
    shift_l = jnp.repeat(shift_c, Wo).reshape(1, lane)

    pass2 = pl.pallas_call(
        functools.partial(_affine_kernel, K=K, Ho=Ho, B=B),
        grid=(N // B,),
        in_specs=[
            pl.BlockSpec((B, Hpad, kin), lambda i: (i, 0, 0)),
            pl.BlockSpec((K, kin, lane), lambda i: (0, 0, 0)),
            pl.BlockSpec((1, lane), lambda i: (0, 0)),
            pl.BlockSpec((1, lane), lambda i: (0, 0)),
        ],
        out_specs=pl.BlockSpec((B, Ho, lane), lambda i: (i, 0, 0)),
        out_shape=jax.ShapeDtypeStruct((N, Ho, lane), jnp.float32),
        compiler_params=pltpu.CompilerParams(
            dimension_semantics=("parallel",)),
        cost_estimate=pl.CostEstimate(
            flops=fl + 3 * N * Ho * lane, transcendentals=0,
            bytes_accessed=(N * Hpad * kin + N * Ho * lane) * itemsize),
    )
    y = pass2(xb, w3, scale_l, shift_l)

    out = y.reshape(N, Ho, Cout, Wo)
    return jnp.transpose(out, (0, 2, 1, 3)).astype(x.dtype)


def kernel(x, weight, bias, gamma, beta):
    del bias
    return _conv_bn_relu(x, weight, gamma, beta, stride=1, padding=1)


# speedup vs baseline: 2.8782x; 1.0166x over previous
import functools

import numpy as np
import jax
import jax.numpy as jnp
from jax.experimental import pallas as pl
from jax.experimental.pallas import tpu as pltpu

_BN_EPS = 1e-5


def _conv_image(xb, w_ref, K, Ho):
    kin = xb.shape[1]
    acc = None
    for kh in range(K):
        a = jax.lax.slice(xb, (kh, 0), (kh + Ho, kin))
        m = jnp.dot(a, w_ref[kh], preferred_element_type=jnp.float32)
        acc = m if acc is None else acc + m
    return acc


def _stats_kernel(xb_ref, w_ref, s_ref, *, K, Ho, B):
    for b in range(B):
        acc = _conv_image(xb_ref[b], w_ref, K, Ho)
        s0 = jnp.sum(acc, axis=0, keepdims=True)
        s1 = jnp.sum(acc * acc, axis=0, keepdims=True)
        s_ref[b] = jnp.concatenate([s0, s1], axis=0)


def _affine_kernel(xb_ref, w_ref, sc_ref, sh_ref, o_ref, *, K, Ho, B):
    for b in range(B):
        acc = _conv_image(xb_ref[b], w_ref, K, Ho)
        o_ref[b] = jnp.maximum(acc * sc_ref[...] + sh_ref[...], 0.0)


def _conv_bn_relu(x, weight, gamma, beta, *, stride, padding):
    N, Cin, H, W = x.shape
    Cout, _, K, _ = weight.shape
    Ho = (H + 2 * padding - K) // stride + 1
    Wo = (W + 2 * padding - K) // stride + 1
    Wp = W + 2 * padding
    lane = Cout * Wo
    kin = Wp * Cin

    rows_needed = (Ho - 1) * stride + K
    Hpad = -(-rows_needed // 16) * 16
    xh = jnp.transpose(x, (0, 2, 3, 1))
    xh = jnp.pad(xh, ((0, 0), (padding, Hpad - H - padding),
                      (padding, padding), (0, 0)))
    xb = xh.reshape(N, Hpad, kin).astype(jnp.bfloat16)

    wt = jnp.transpose(weight, (2, 3, 1, 0))
    kw_i, ci_i, wo_i, co_i = np.meshgrid(
        np.arange(K), np.arange(Cin), np.arange(Wo), np.arange(Cout),
        indexing="ij")
    rows = (wo_i * stride + kw_i) * Cin + ci_i
    cols = co_i * Wo + wo_i
    w3 = jnp.zeros((K, kin, lane), jnp.float32)
    w3 = w3.at[:, rows, cols].set(wt[:, kw_i, ci_i, co_i]).astype(jnp.bfloat16)

    fl = 2 * N * K * Ho * kin * lane
    itemsize = 2
    B = 16 if N % 16 == 0 else (8 if N % 8 == 0 else 1)

    pass1 = pl.pallas_call(
        functools.partial(_stats_kernel, K=K, Ho=Ho, B=B),
        grid=(N // B,),
        in_specs=[
            pl.BlockSpec((B, Hpad, kin), lambda i: (i, 0, 0)),
            pl.BlockSpec((K, kin, lane), lambda i: (0, 0, 0)),
        ],
        out_specs=pl.BlockSpec((B, 2, lane), lambda i: (i, 0, 0)),
        out_shape=jax.ShapeDtypeStruct((N, 2, lane), jnp.float32),
        compiler_params=pltpu.CompilerParams(
            dimension_semantics=("parallel",)),
        cost_estimate=pl.CostEstimate(
            flops=fl, transcendentals=0,
            bytes_accessed=N * Hpad * kin * itemsize + N * 2 * lane * 4),
    )
    part = pass1(xb, w3)

    Mtot = N * Ho * Wo
    st = part.reshape(N, 2, Cout, Wo).sum(axis=(0, 3))
    mean = st[0] / Mtot
    var = st[1] / Mtot - mean * mean
    scale_c = gamma.astype(jnp.float32) * jax.lax.rsqrt(var + _BN_EPS)
    shift_c = beta.astype(jnp.float32) - mean * scale_c
    scale_l = jnp.repeat(scale_c, Wo).reshape(1, lane)
    shift_l = jnp.repeat(shift_c, Wo).reshape(1, lane)

    pass2 = pl.pallas_call(
        functools.partial(_affine_kernel, K=K, Ho=Ho, B=B),
        grid=(N // B,),
        in_specs=[
            pl.BlockSpec((B, Hpad, kin), lambda i: (i, 0, 0)),
            pl.BlockSpec((K, kin, lane), lambda i: (0, 0, 0)),
            pl.BlockSpec((1, lane), lambda i: (0, 0)),
            pl.BlockSpec((1, lane), lambda i: (0, 0)),
        ],
        out_specs=pl.BlockSpec((B, Ho, lane), lambda i: (i, 0, 0)),
        out_shape=jax.ShapeDtypeStruct((N, Ho, lane), jnp.float32),
        compiler_params=pltpu.CompilerParams(
            dimension_semantics=("parallel",)),
        cost_estimate=pl.CostEstimate(
            flops=fl + 3 * N * Ho * lane, transcendentals=0,
            bytes_accessed=(N * Hpad * kin + N * Ho * lane) * itemsize),
    )
    y = pass2(xb, w3, scale_l, shift_l)

    out = y.reshape(N, Ho, Cout, Wo)
    return jnp.transpose(out, (0, 2, 1, 3)).astype(x.dtype)


def kernel(x, weight, bias, gamma, beta):
    del bias
    return _conv_bn_relu(x, weight, gamma, beta, stride=1, padding=1)
